# Initial kernel scaffold; baseline (speedup 1.0000x reference)
import functools

import jax
import jax.numpy as jnp
from jax import lax
from jax.experimental import pallas as pl
from jax.experimental.pallas import tpu as pltpu
from jax.experimental.pallas import tpu_sc as plsc

N_NODES = 100000
N_EDGES = 1600000
IN_C, HID_C, OUT_C = 12, 128, 64

NC, NS = 2, 16
CW = 16
NPAD = 100352
RPT = NPAD // NS
ER = 12544
EPAD = ER * 128
IDXB = 8

_mesh = plsc.VectorSubcoreMesh(core_axis_name="c", subcore_axis_name="s")


def _init_bufs(zbuf, fill_v, fill):
    @pl.loop(0, 128)
    def _(i):
        zbuf[i, :] = jnp.zeros((CW,), jnp.float32)
        if fill_v is not None:
            fill_v[i, :] = jnp.full((CW,), fill, jnp.float32)


def _zero_acc(zbuf, acc, sid):
    @pl.loop(0, RPT // 128)
    def _(b):
        pltpu.sync_copy(zbuf, acc.at[pl.ds(sid * RPT + b * 128, 128)])


@functools.partial(
    pl.kernel,
    mesh=_mesh,
    out_type=jax.ShapeDtypeStruct((NC, NPAD, CW), jnp.float32),
    scratch_types=[
        pltpu.VMEM((IDXB, 128), jnp.int32),
        pltpu.VMEM((128, CW), jnp.float32),
        pltpu.VMEM((128, CW), jnp.float32),
        pltpu.VMEM_SHARED((NPAD, CW), jnp.float32),
    ],
)
def _sc_degree(dst_hbm, out_hbm, didx, ones_v, zbuf, acc):
    cid = lax.axis_index("c")
    sid = lax.axis_index("s")
    _init_bufs(zbuf, ones_v, 1.0)
    _zero_acc(zbuf, acc, sid)
    plsc.subcore_barrier()
    base = (cid * NS + sid) * (ER // (NC * NS))

    @pl.loop(0, ER // (NC * NS) // IDXB)
    def _(g):
        pltpu.sync_copy(dst_hbm.at[pl.ds(base + g * IDXB, IDXB)], didx)
        for j in range(IDXB):
            pltpu.sync_copy(ones_v, acc.at[didx.at[j]], add=True)

    plsc.subcore_barrier()
    pltpu.sync_copy(
        acc.at[pl.ds(sid * RPT, RPT)], out_hbm.at[cid, pl.ds(sid * RPT, RPT)]
    )


@functools.partial(
    pl.kernel,
    mesh=_mesh,
    out_type=jax.ShapeDtypeStruct((NC, NPAD, CW), jnp.float32),
    scratch_types=[
        pltpu.VMEM((IDXB, 128), jnp.int32),
        pltpu.VMEM((IDXB, 128), jnp.int32),
        pltpu.VMEM((128, CW), jnp.float32),
        pltpu.VMEM((128, CW), jnp.float32),
        pltpu.VMEM_SHARED((NPAD, CW), jnp.float32),
    ],
)
def _sc_prop16(src_hbm, dst_hbm, tab_hbm, out_hbm, sidx, didx, rows, zbuf, acc):
    cid = lax.axis_index("c")
    sid = lax.axis_index("s")
    _init_bufs(zbuf, None, 0.0)
    _zero_acc(zbuf, acc, sid)
    plsc.subcore_barrier()
    base = (cid * NS + sid) * (ER // (NC * NS))

    @pl.loop(0, ER // (NC * NS) // IDXB)
    def _(g):
        pltpu.sync_copy(src_hbm.at[pl.ds(base + g * IDXB, IDXB)], sidx)
        pltpu.sync_copy(dst_hbm.at[pl.ds(base + g * IDXB, IDXB)], didx)
        for j in range(IDXB):
            pltpu.sync_copy(tab_hbm.at[sidx.at[j]], rows)
            pltpu.sync_copy(rows, acc.at[didx.at[j]], add=True)

    plsc.subcore_barrier()
    pltpu.sync_copy(
        acc.at[pl.ds(sid * RPT, RPT)], out_hbm.at[cid, pl.ds(sid * RPT, RPT)]
    )


@functools.partial(
    pl.kernel,
    mesh=_mesh,
    out_type=jax.ShapeDtypeStruct((4, NPAD, CW), jnp.float32),
    scratch_types=[
        pltpu.VMEM((IDXB, 128), jnp.int32),
        pltpu.VMEM((IDXB, 128), jnp.int32),
        pltpu.VMEM((128, CW), jnp.float32),
        pltpu.VMEM((128, CW), jnp.float32),
        pltpu.VMEM_SHARED((NPAD, CW), jnp.float32),
    ],
)
def _sc_prop64(src_hbm, dst_hbm, tab_hbm, out_hbm, sidx, didx, rows, zbuf, acc):
    cid = lax.axis_index("c")
    sid = lax.axis_index("s")
    _init_bufs(zbuf, None, 0.0)

    def chunk_body(tab, outk):
        _zero_acc(zbuf, acc, sid)
        plsc.subcore_barrier()
        base = sid * (ER // NS)

        @pl.loop(0, ER // NS // IDXB)
        def _(g):
            pltpu.sync_copy(src_hbm.at[pl.ds(base + g * IDXB, IDXB)], sidx)
            pltpu.sync_copy(dst_hbm.at[pl.ds(base + g * IDXB, IDXB)], didx)
            for j in range(IDXB):
                pltpu.sync_copy(tab.at[sidx.at[j]], rows)
                pltpu.sync_copy(rows, acc.at[didx.at[j]], add=True)

        plsc.subcore_barrier()
        pltpu.sync_copy(acc.at[pl.ds(sid * RPT, RPT)], outk.at[pl.ds(sid * RPT, RPT)])
        plsc.subcore_barrier()

    @pl.when(cid == 0)
    def _():
        chunk_body(tab_hbm.at[0], out_hbm.at[0])
        chunk_body(tab_hbm.at[1], out_hbm.at[1])

    @pl.when(cid == 1)
    def _():
        chunk_body(tab_hbm.at[2], out_hbm.at[2])
        chunk_body(tab_hbm.at[3], out_hbm.at[3])


BL1 = 1024


def _tc1_body(degp_ref, x_ref, dinv_ref, g1_ref):
    deg = degp_ref[0][:, 0:1] + degp_ref[1][:, 0:1] + 1.0
    dv = lax.rsqrt(deg)
    dinv_ref[...] = dv
    g1_ref[...] = x_ref[...] * dv


def _tc2_body(s1p_ref, g1_ref, dinv_ref, w1_ref, b1_ref, w2_ref, g2_ref):
    dv = dinv_ref[...]
    z = (s1p_ref[0] + s1p_ref[1] + g1_ref[...]) * dv
    h = jnp.dot(z, w1_ref[...], preferred_element_type=jnp.float32,
                precision=lax.Precision.HIGHEST)
    h = jnp.maximum(h + b1_ref[...], 0.0)
    g2 = jnp.dot(h, w2_ref[...], preferred_element_type=jnp.float32,
                 precision=lax.Precision.HIGHEST) * dv
    for c in range(4):
        g2_ref[c] = g2[:, c * CW:(c + 1) * CW]


BL3 = 1000


def _tc3_body(s2_ref, g2_ref, dinv_ref, b2_ref, out_ref):
    dv = dinv_ref[...]
    parts = [(s2_ref[c] + g2_ref[c]) * dv for c in range(4)]
    out_ref[...] = jnp.concatenate(parts, axis=1) + b2_ref[...]


def kernel(x, edge_index, W1, b1, W2, b2):
    src = edge_index[0]
    dst = edge_index[1]
    pad = EPAD - N_EDGES
    src2d = jnp.concatenate(
        [src, jnp.full((pad,), N_NODES, jnp.int32)]).reshape(ER, 128)
    dst2d = jnp.concatenate(
        [dst, jnp.full((pad,), N_NODES, jnp.int32)]).reshape(ER, 128)
    x_pad = jnp.pad(x, ((0, NPAD - N_NODES), (0, CW - IN_C)))
    w1_pad = jnp.pad(W1, ((0, CW - IN_C), (0, 0)))
    b1r = b1.reshape(1, HID_C)
    b2r = b2.reshape(1, OUT_C)

    degp = _sc_degree(dst2d)

    dinv, g1 = pl.pallas_call(
        _tc1_body,
        grid=(NPAD // BL1,),
        in_specs=[
            pl.BlockSpec((NC, BL1, CW), lambda i: (0, i, 0)),
            pl.BlockSpec((BL1, CW), lambda i: (i, 0)),
        ],
        out_specs=[
            pl.BlockSpec((BL1, 1), lambda i: (i, 0)),
            pl.BlockSpec((BL1, CW), lambda i: (i, 0)),
        ],
        out_shape=[
            jax.ShapeDtypeStruct((NPAD, 1), jnp.float32),
            jax.ShapeDtypeStruct((NPAD, CW), jnp.float32),
        ],
    )(degp, x_pad)

    s1p = _sc_prop16(src2d, dst2d, g1)

    g2 = pl.pallas_call(
        _tc2_body,
        grid=(NPAD // BL1,),
        in_specs=[
            pl.BlockSpec((NC, BL1, CW), lambda i: (0, i, 0)),
            pl.BlockSpec((BL1, CW), lambda i: (i, 0)),
            pl.BlockSpec((BL1, 1), lambda i: (i, 0)),
            pl.BlockSpec((CW, HID_C), lambda i: (0, 0)),
            pl.BlockSpec((1, HID_C), lambda i: (0, 0)),
            pl.BlockSpec((HID_C, OUT_C), lambda i: (0, 0)),
        ],
        out_specs=pl.BlockSpec((4, BL1, CW), lambda i: (0, i, 0)),
        out_shape=jax.ShapeDtypeStruct((4, NPAD, CW), jnp.float32),
    )(s1p, g1, dinv, w1_pad, b1r, W2)

    s2 = _sc_prop64(src2d, dst2d, g2)

    out = pl.pallas_call(
        _tc3_body,
        grid=(N_NODES // BL3,),
        in_specs=[
            pl.BlockSpec((4, BL3, CW), lambda i: (0, i, 0)),
            pl.BlockSpec((4, BL3, CW), lambda i: (0, i, 0)),
            pl.BlockSpec((BL3, 1), lambda i: (i, 0)),
            pl.BlockSpec((1, OUT_C), lambda i: (0, 0)),
        ],
        out_specs=pl.BlockSpec((BL3, OUT_C), lambda i: (i, 0)),
        out_shape=jax.ShapeDtypeStruct((N_NODES, OUT_C), jnp.float32),
    )(s2, g2, dinv, b2r)

    return out


# speedup vs baseline: 13.6359x; 13.6359x over previous
import functools

import jax
import jax.numpy as jnp
from jax import lax
from jax.experimental import pallas as pl
from jax.experimental.pallas import tpu as pltpu
from jax.experimental.pallas import tpu_sc as plsc

N_NODES = 100000
N_EDGES = 1600000
IN_C, HID_C, OUT_C = 12, 128, 64

NC, NS = 2, 16
CW = 16
NPAD = 100352
RPT = NPAD // NS
ER = 12544
EPAD = ER * 128
IDXB = 8

_mesh = plsc.VectorSubcoreMesh(core_axis_name="c", subcore_axis_name="s")
_sc_params = pltpu.CompilerParams(use_tc_tiling_on_sc=False)


@functools.partial(
    pl.kernel,
    mesh=_mesh,
    compiler_params=_sc_params,
    out_type=jax.ShapeDtypeStruct((NC, NPAD, CW), jnp.float32),
    scratch_types=[
        pltpu.VMEM((IDXB, 128), jnp.int32),
        pltpu.VMEM((128, CW), jnp.float32),
        pltpu.VMEM_SHARED((NPAD, CW), jnp.float32),
    ],
)
def _sc_degree(dst_hbm, zero_hbm, out_hbm, didx, ones_v, acc):
    cid = lax.axis_index("c")
    sid = lax.axis_index("s")

    @pl.loop(0, 128)
    def _(i):
        ones_v[i, :] = jnp.full((CW,), 1.0, jnp.float32)

    pltpu.sync_copy(zero_hbm.at[pl.ds(sid * RPT, RPT)],
                    acc.at[pl.ds(sid * RPT, RPT)])
    plsc.subcore_barrier()
    base = (cid * NS + sid) * (ER // (NC * NS))

    @pl.loop(0, ER // (NC * NS) // IDXB)
    def _(g):
        pltpu.sync_copy(dst_hbm.at[pl.ds(base + g * IDXB, IDXB)], didx)
        for j in range(IDXB):
            pltpu.sync_copy(ones_v, acc.at[didx.at[j]], add=True)

    plsc.subcore_barrier()
    pltpu.sync_copy(
        acc.at[pl.ds(sid * RPT, RPT)], out_hbm.at[cid, pl.ds(sid * RPT, RPT)]
    )


@functools.partial(
    pl.kernel,
    mesh=_mesh,
    compiler_params=_sc_params,
    out_type=jax.ShapeDtypeStruct((NC, NPAD, CW), jnp.float32),
    scratch_types=[
        pltpu.VMEM((IDXB, 128), jnp.int32),
        pltpu.VMEM((IDXB, 128), jnp.int32),
        pltpu.VMEM((128, CW), jnp.float32),
        pltpu.VMEM_SHARED((NPAD, CW), jnp.float32),
    ],
)
def _sc_prop16(src_hbm, dst_hbm, tab_hbm, zero_hbm, out_hbm,
               sidx, didx, rows, acc):
    cid = lax.axis_index("c")
    sid = lax.axis_index("s")
    pltpu.sync_copy(zero_hbm.at[pl.ds(sid * RPT, RPT)],
                    acc.at[pl.ds(sid * RPT, RPT)])
    plsc.subcore_barrier()
    base = (cid * NS + sid) * (ER // (NC * NS))

    @pl.loop(0, ER // (NC * NS) // IDXB)
    def _(g):
        pltpu.sync_copy(src_hbm.at[pl.ds(base + g * IDXB, IDXB)], sidx)
        pltpu.sync_copy(dst_hbm.at[pl.ds(base + g * IDXB, IDXB)], didx)
        for j in range(IDXB):
            pltpu.sync_copy(tab_hbm.at[sidx.at[j]], rows)
            pltpu.sync_copy(rows, acc.at[didx.at[j]], add=True)

    plsc.subcore_barrier()
    pltpu.sync_copy(
        acc.at[pl.ds(sid * RPT, RPT)], out_hbm.at[cid, pl.ds(sid * RPT, RPT)]
    )


@functools.partial(
    pl.kernel,
    mesh=_mesh,
    compiler_params=_sc_params,
    out_type=jax.ShapeDtypeStruct((4, NPAD, CW), jnp.float32),
    scratch_types=[
        pltpu.VMEM((IDXB, 128), jnp.int32),
        pltpu.VMEM((IDXB, 128), jnp.int32),
        pltpu.VMEM((128, CW), jnp.float32),
        pltpu.VMEM_SHARED((NPAD, CW), jnp.float32),
    ],
)
def _sc_prop64(src_hbm, dst_hbm, tab_hbm, zero_hbm, out_hbm,
               sidx, didx, rows, acc):
    cid = lax.axis_index("c")
    sid = lax.axis_index("s")

    def chunk_body(tab, outk):
        pltpu.sync_copy(zero_hbm.at[pl.ds(sid * RPT, RPT)],
                        acc.at[pl.ds(sid * RPT, RPT)])
        plsc.subcore_barrier()
        base = sid * (ER // NS)

        @pl.loop(0, ER // NS // IDXB)
        def _(g):
            pltpu.sync_copy(src_hbm.at[pl.ds(base + g * IDXB, IDXB)], sidx)
            pltpu.sync_copy(dst_hbm.at[pl.ds(base + g * IDXB, IDXB)], didx)
            for j in range(IDXB):
                pltpu.sync_copy(tab.at[sidx.at[j]], rows)
                pltpu.sync_copy(rows, acc.at[didx.at[j]], add=True)

        plsc.subcore_barrier()
        pltpu.sync_copy(acc.at[pl.ds(sid * RPT, RPT)],
                        outk.at[pl.ds(sid * RPT, RPT)])
        plsc.subcore_barrier()

    @pl.when(cid == 0)
    def _():
        chunk_body(tab_hbm.at[0], out_hbm.at[0])
        chunk_body(tab_hbm.at[1], out_hbm.at[1])

    @pl.when(cid == 1)
    def _():
        chunk_body(tab_hbm.at[2], out_hbm.at[2])
        chunk_body(tab_hbm.at[3], out_hbm.at[3])


BL1 = 1024


def _tc1_body(degp_ref, x_ref, dinv_ref, g1_ref):
    deg = degp_ref[0][:, 0:1] + degp_ref[1][:, 0:1] + 1.0
    dv = lax.rsqrt(deg)
    dinv_ref[...] = dv
    g1_ref[...] = x_ref[...] * dv


def _tc2_body(s1p_ref, g1_ref, dinv_ref, w1_ref, b1_ref, w2_ref, g2_ref):
    dv = dinv_ref[...]
    z = (s1p_ref[0] + s1p_ref[1] + g1_ref[...]) * dv
    h = jnp.dot(z, w1_ref[...], preferred_element_type=jnp.float32,
                precision=lax.Precision.HIGHEST)
    h = jnp.maximum(h + b1_ref[...], 0.0)
    g2 = jnp.dot(h, w2_ref[...], preferred_element_type=jnp.float32,
                 precision=lax.Precision.HIGHEST) * dv
    for c in range(4):
        g2_ref[c] = g2[:, c * CW:(c + 1) * CW]


BL3 = 1000


def _tc3_body(s2_ref, g2_ref, dinv_ref, b2_ref, out_ref):
    dv = dinv_ref[...]
    parts = [(s2_ref[c] + g2_ref[c]) * dv for c in range(4)]
    out_ref[...] = jnp.concatenate(parts, axis=1) + b2_ref[...]


def kernel(x, edge_index, W1, b1, W2, b2):
    src = edge_index[0]
    dst = edge_index[1]
    pad = EPAD - N_EDGES
    pad_idx = N_NODES + (jnp.arange(pad, dtype=jnp.int32) % (NPAD - N_NODES))
    src2d = jnp.concatenate([src, pad_idx]).reshape(ER, 128)
    dst2d = jnp.concatenate([dst, pad_idx]).reshape(ER, 128)
    x_pad = jnp.pad(x, ((0, NPAD - N_NODES), (0, CW - IN_C)))
    w1_pad = jnp.pad(W1, ((0, CW - IN_C), (0, 0)))
    b1r = b1.reshape(1, HID_C)
    b2r = b2.reshape(1, OUT_C)
    zeros_n = jnp.zeros((NPAD, CW), jnp.float32)

    degp = _sc_degree(dst2d, zeros_n)

    dinv, g1 = pl.pallas_call(
        _tc1_body,
        grid=(NPAD // BL1,),
        in_specs=[
            pl.BlockSpec((NC, BL1, CW), lambda i: (0, i, 0)),
            pl.BlockSpec((BL1, CW), lambda i: (i, 0)),
        ],
        out_specs=[
            pl.BlockSpec((BL1, 1), lambda i: (i, 0)),
            pl.BlockSpec((BL1, CW), lambda i: (i, 0)),
        ],
        out_shape=[
            jax.ShapeDtypeStruct((NPAD, 1), jnp.float32),
            jax.ShapeDtypeStruct((NPAD, CW), jnp.float32),
        ],
    )(degp, x_pad)

    s1p = _sc_prop16(src2d, dst2d, g1, zeros_n)

    g2 = pl.pallas_call(
        _tc2_body,
        grid=(NPAD // BL1,),
        in_specs=[
            pl.BlockSpec((NC, BL1, CW), lambda i: (0, i, 0)),
            pl.BlockSpec((BL1, CW), lambda i: (i, 0)),
            pl.BlockSpec((BL1, 1), lambda i: (i, 0)),
            pl.BlockSpec((CW, HID_C), lambda i: (0, 0)),
            pl.BlockSpec((1, HID_C), lambda i: (0, 0)),
            pl.BlockSpec((HID_C, OUT_C), lambda i: (0, 0)),
        ],
        out_specs=pl.BlockSpec((4, BL1, CW), lambda i: (0, i, 0)),
        out_shape=jax.ShapeDtypeStruct((4, NPAD, CW), jnp.float32),
    )(s1p, g1, dinv, w1_pad, b1r, W2)

    s2 = _sc_prop64(src2d, dst2d, g2, zeros_n)

    out = pl.pallas_call(
        _tc3_body,
        grid=(N_NODES // BL3,),
        in_specs=[
            pl.BlockSpec((4, BL3, CW), lambda i: (0, i, 0)),
            pl.BlockSpec((4, BL3, CW), lambda i: (0, i, 0)),
            pl.BlockSpec((BL3, 1), lambda i: (i, 0)),
            pl.BlockSpec((1, OUT_C), lambda i: (0, 0)),
        ],
        out_specs=pl.BlockSpec((BL3, OUT_C), lambda i: (i, 0)),
        out_shape=jax.ShapeDtypeStruct((N_NODES, OUT_C), jnp.float32),
    )(s2, g2, dinv, b2r)

    return out


# speedup vs baseline: 18.4033x; 1.3496x over previous
import functools

import jax
import jax.numpy as jnp
from jax import lax
from jax.experimental import pallas as pl
from jax.experimental.pallas import tpu as pltpu
from jax.experimental.pallas import tpu_sc as plsc

N_NODES = 100000
N_EDGES = 1600000
IN_C, HID_C, OUT_C = 12, 128, 64

NC, NS = 2, 16
CW = 16
NPAD = 100352
RPT = NPAD // NS
ER = 12544
EPAD = ER * 128
IDXB = 8

_mesh = plsc.VectorSubcoreMesh(core_axis_name="c", subcore_axis_name="s")
_sc_params = pltpu.CompilerParams(use_tc_tiling_on_sc=False)


@functools.partial(
    pl.kernel,
    mesh=_mesh,
    compiler_params=_sc_params,
    out_type=jax.ShapeDtypeStruct((NC, NPAD, CW), jnp.float32),
    scratch_types=[
        pltpu.VMEM((IDXB, 128), jnp.int32),
        pltpu.VMEM((128, CW), jnp.float32),
        pltpu.VMEM_SHARED((NPAD, CW), jnp.float32),
        pltpu.SemaphoreType.DMA,
    ],
)
def _sc_degree(dst_hbm, zero_hbm, out_hbm, didx, ones_v, acc, sem):
    cid = lax.axis_index("c")
    sid = lax.axis_index("s")

    @pl.loop(0, 128)
    def _(i):
        ones_v[i, :] = jnp.full((CW,), 1.0, jnp.float32)

    pltpu.sync_copy(zero_hbm.at[pl.ds(sid * RPT, RPT)],
                    acc.at[pl.ds(sid * RPT, RPT)])
    plsc.subcore_barrier()
    base = (cid * NS + sid) * (ER // (NC * NS))

    @pl.loop(0, ER // (NC * NS) // IDXB)
    def _(g):
        pltpu.sync_copy(dst_hbm.at[pl.ds(base + g * IDXB, IDXB)], didx)
        descs = [pltpu.async_copy(ones_v, acc.at[didx.at[j]], sem, add=True)
                 for j in range(IDXB)]
        for d in descs:
            d.wait()

    plsc.subcore_barrier()
    pltpu.sync_copy(
        acc.at[pl.ds(sid * RPT, RPT)], out_hbm.at[cid, pl.ds(sid * RPT, RPT)]
    )


def _edge_block(tab, acc, sidx, didx, rows2, sg, ss):
    gd = [None] * IDXB
    sd = [None] * IDXB
    gd[0] = pltpu.async_copy(tab.at[sidx.at[0]], rows2[0], sg[0])
    for j in range(IDXB):
        if j + 1 < IDXB:
            if j >= 1:
                sd[j - 1].wait()
            gd[j + 1] = pltpu.async_copy(
                tab.at[sidx.at[j + 1]], rows2[(j + 1) % 2], sg[(j + 1) % 2])
        gd[j].wait()
        sd[j] = pltpu.async_copy(rows2[j % 2], acc.at[didx.at[j]],
                                 ss[j % 2], add=True)
    sd[IDXB - 2].wait()
    sd[IDXB - 1].wait()


@functools.partial(
    pl.kernel,
    mesh=_mesh,
    compiler_params=_sc_params,
    out_type=jax.ShapeDtypeStruct((NC, NPAD, CW), jnp.float32),
    scratch_types=[
        pltpu.VMEM((IDXB, 128), jnp.int32),
        pltpu.VMEM((IDXB, 128), jnp.int32),
        pltpu.VMEM((128, CW), jnp.float32),
        pltpu.VMEM((128, CW), jnp.float32),
        pltpu.VMEM_SHARED((NPAD, CW), jnp.float32),
        pltpu.SemaphoreType.DMA,
        pltpu.SemaphoreType.DMA,
        pltpu.SemaphoreType.DMA,
        pltpu.SemaphoreType.DMA,
    ],
)
def _sc_prop16(src_hbm, dst_hbm, tab_hbm, zero_hbm, out_hbm,
               sidx, didx, rows_a, rows_b, acc, sg0, sg1, ss0, ss1):
    cid = lax.axis_index("c")
    sid = lax.axis_index("s")
    pltpu.sync_copy(zero_hbm.at[pl.ds(sid * RPT, RPT)],
                    acc.at[pl.ds(sid * RPT, RPT)])
    plsc.subcore_barrier()
    base = (cid * NS + sid) * (ER // (NC * NS))

    @pl.loop(0, ER // (NC * NS) // IDXB)
    def _(g):
        pltpu.sync_copy(src_hbm.at[pl.ds(base + g * IDXB, IDXB)], sidx)
        pltpu.sync_copy(dst_hbm.at[pl.ds(base + g * IDXB, IDXB)], didx)
        _edge_block(tab_hbm, acc, sidx, didx, (rows_a, rows_b),
                    (sg0, sg1), (ss0, ss1))

    plsc.subcore_barrier()
    pltpu.sync_copy(
        acc.at[pl.ds(sid * RPT, RPT)], out_hbm.at[cid, pl.ds(sid * RPT, RPT)]
    )


@functools.partial(
    pl.kernel,
    mesh=_mesh,
    compiler_params=_sc_params,
    out_type=jax.ShapeDtypeStruct((4, NPAD, CW), jnp.float32),
    scratch_types=[
        pltpu.VMEM((IDXB, 128), jnp.int32),
        pltpu.VMEM((IDXB, 128), jnp.int32),
        pltpu.VMEM((128, CW), jnp.float32),
        pltpu.VMEM((128, CW), jnp.float32),
        pltpu.VMEM_SHARED((NPAD, CW), jnp.float32),
        pltpu.SemaphoreType.DMA,
        pltpu.SemaphoreType.DMA,
        pltpu.SemaphoreType.DMA,
        pltpu.SemaphoreType.DMA,
    ],
)
def _sc_prop64(src_hbm, dst_hbm, tab_hbm, zero_hbm, out_hbm,
               sidx, didx, rows_a, rows_b, acc, sg0, sg1, ss0, ss1):
    cid = lax.axis_index("c")
    sid = lax.axis_index("s")

    def chunk_body(tab, outk):
        pltpu.sync_copy(zero_hbm.at[pl.ds(sid * RPT, RPT)],
                        acc.at[pl.ds(sid * RPT, RPT)])
        plsc.subcore_barrier()
        base = sid * (ER // NS)

        @pl.loop(0, ER // NS // IDXB)
        def _(g):
            pltpu.sync_copy(src_hbm.at[pl.ds(base + g * IDXB, IDXB)], sidx)
            pltpu.sync_copy(dst_hbm.at[pl.ds(base + g * IDXB, IDXB)], didx)
            _edge_block(tab, acc, sidx, didx, (rows_a, rows_b),
                        (sg0, sg1), (ss0, ss1))

        plsc.subcore_barrier()
        pltpu.sync_copy(acc.at[pl.ds(sid * RPT, RPT)],
                        outk.at[pl.ds(sid * RPT, RPT)])
        plsc.subcore_barrier()

    @pl.when(cid == 0)
    def _():
        chunk_body(tab_hbm.at[0], out_hbm.at[0])
        chunk_body(tab_hbm.at[1], out_hbm.at[1])

    @pl.when(cid == 1)
    def _():
        chunk_body(tab_hbm.at[2], out_hbm.at[2])
        chunk_body(tab_hbm.at[3], out_hbm.at[3])


BL1 = 1024


def _tc1_body(degp_ref, x_ref, dinv_ref, g1_ref):
    deg = degp_ref[0][:, 0:1] + degp_ref[1][:, 0:1] + 1.0
    dv = lax.rsqrt(deg)
    dinv_ref[...] = dv
    g1_ref[...] = x_ref[...] * dv


def _tc2_body(s1p_ref, g1_ref, dinv_ref, w1_ref, b1_ref, w2_ref, g2_ref):
    dv = dinv_ref[...]
    z = (s1p_ref[0] + s1p_ref[1] + g1_ref[...]) * dv
    h = jnp.dot(z, w1_ref[...], preferred_element_type=jnp.float32,
                precision=lax.Precision.HIGHEST)
    h = jnp.maximum(h + b1_ref[...], 0.0)
    g2 = jnp.dot(h, w2_ref[...], preferred_element_type=jnp.float32,
                 precision=lax.Precision.HIGHEST) * dv
    for c in range(4):
        g2_ref[c] = g2[:, c * CW:(c + 1) * CW]


BL3 = 1000


def _tc3_body(s2_ref, g2_ref, dinv_ref, b2_ref, out_ref):
    dv = dinv_ref[...]
    parts = [(s2_ref[c] + g2_ref[c]) * dv for c in range(4)]
    out_ref[...] = jnp.concatenate(parts, axis=1) + b2_ref[...]


def kernel(x, edge_index, W1, b1, W2, b2):
    src = edge_index[0]
    dst = edge_index[1]
    pad = EPAD - N_EDGES
    pad_idx = N_NODES + (jnp.arange(pad, dtype=jnp.int32) % (NPAD - N_NODES))
    src2d = jnp.concatenate([src, pad_idx]).reshape(ER, 128)
    dst2d = jnp.concatenate([dst, pad_idx]).reshape(ER, 128)
    x_pad = jnp.pad(x, ((0, NPAD - N_NODES), (0, CW - IN_C)))
    w1_pad = jnp.pad(W1, ((0, CW - IN_C), (0, 0)))
    b1r = b1.reshape(1, HID_C)
    b2r = b2.reshape(1, OUT_C)
    zeros_n = jnp.zeros((NPAD, CW), jnp.float32)

    degp = _sc_degree(dst2d, zeros_n)

    dinv, g1 = pl.pallas_call(
        _tc1_body,
        grid=(NPAD // BL1,),
        in_specs=[
            pl.BlockSpec((NC, BL1, CW), lambda i: (0, i, 0)),
            pl.BlockSpec((BL1, CW), lambda i: (i, 0)),
        ],
        out_specs=[
            pl.BlockSpec((BL1, 1), lambda i: (i, 0)),
            pl.BlockSpec((BL1, CW), lambda i: (i, 0)),
        ],
        out_shape=[
            jax.ShapeDtypeStruct((NPAD, 1), jnp.float32),
            jax.ShapeDtypeStruct((NPAD, CW), jnp.float32),
        ],
    )(degp, x_pad)

    s1p = _sc_prop16(src2d, dst2d, g1, zeros_n)

    g2 = pl.pallas_call(
        _tc2_body,
        grid=(NPAD // BL1,),
        in_specs=[
            pl.BlockSpec((NC, BL1, CW), lambda i: (0, i, 0)),
            pl.BlockSpec((BL1, CW), lambda i: (i, 0)),
            pl.BlockSpec((BL1, 1), lambda i: (i, 0)),
            pl.BlockSpec((CW, HID_C), lambda i: (0, 0)),
            pl.BlockSpec((1, HID_C), lambda i: (0, 0)),
            pl.BlockSpec((HID_C, OUT_C), lambda i: (0, 0)),
        ],
        out_specs=pl.BlockSpec((4, BL1, CW), lambda i: (0, i, 0)),
        out_shape=jax.ShapeDtypeStruct((4, NPAD, CW), jnp.float32),
    )(s1p, g1, dinv, w1_pad, b1r, W2)

    s2 = _sc_prop64(src2d, dst2d, g2, zeros_n)

    out = pl.pallas_call(
        _tc3_body,
        grid=(N_NODES // BL3,),
        in_specs=[
            pl.BlockSpec((4, BL3, CW), lambda i: (0, i, 0)),
            pl.BlockSpec((4, BL3, CW), lambda i: (0, i, 0)),
            pl.BlockSpec((BL3, 1), lambda i: (i, 0)),
            pl.BlockSpec((1, OUT_C), lambda i: (0, 0)),
        ],
        out_specs=pl.BlockSpec((BL3, OUT_C), lambda i: (i, 0)),
        out_shape=jax.ShapeDtypeStruct((N_NODES, OUT_C), jnp.float32),
    )(s2, g2, dinv, b2r)

    return out


# speedup vs baseline: 20.9393x; 1.1378x over previous
import functools

import jax
import jax.numpy as jnp
from jax import lax
from jax.experimental import pallas as pl
from jax.experimental.pallas import tpu as pltpu
from jax.experimental.pallas import tpu_sc as plsc

N_NODES = 100000
N_EDGES = 1600000
IN_C, HID_C, OUT_C = 12, 128, 64

NC, NS = 2, 16
CW = 16
NPAD = 100352
RPT = NPAD // NS
ER = 12544
EPAD = ER * 128
IDXB = 8

_mesh = plsc.VectorSubcoreMesh(core_axis_name="c", subcore_axis_name="s")
_sc_params = pltpu.CompilerParams(use_tc_tiling_on_sc=False)


@functools.partial(
    pl.kernel,
    mesh=_mesh,
    compiler_params=_sc_params,
    out_type=jax.ShapeDtypeStruct((NC, NPAD, CW), jnp.float32),
    scratch_types=[
        pltpu.VMEM((IDXB, 128), jnp.int32),
        pltpu.VMEM((128, CW), jnp.float32),
        pltpu.VMEM_SHARED((NPAD, CW), jnp.float32),
        pltpu.SemaphoreType.DMA,
    ],
)
def _sc_degree(dst_hbm, zero_hbm, out_hbm, didx, ones_v, acc, sem):
    cid = lax.axis_index("c")
    sid = lax.axis_index("s")

    @pl.loop(0, 128)
    def _(i):
        ones_v[i, :] = jnp.full((CW,), 1.0, jnp.float32)

    pltpu.sync_copy(zero_hbm.at[pl.ds(sid * RPT, RPT)],
                    acc.at[pl.ds(sid * RPT, RPT)])
    plsc.subcore_barrier()
    base = (cid * NS + sid) * (ER // (NC * NS))

    @pl.loop(0, ER // (NC * NS) // IDXB)
    def _(g):
        pltpu.sync_copy(dst_hbm.at[pl.ds(base + g * IDXB, IDXB)], didx)
        descs = [pltpu.async_copy(ones_v, acc.at[didx.at[j]], sem, add=True)
                 for j in range(IDXB)]
        for d in descs:
            d.wait()

    plsc.subcore_barrier()
    pltpu.sync_copy(
        acc.at[pl.ds(sid * RPT, RPT)], out_hbm.at[cid, pl.ds(sid * RPT, RPT)]
    )


def _edge_block(tab, acc, sidx, didx, rows2, sg, ss):
    gd = [None] * IDXB
    sd = [None] * IDXB
    gd[0] = pltpu.async_copy(tab.at[sidx.at[0]], rows2[0], sg[0])
    for j in range(IDXB):
        if j + 1 < IDXB:
            if j >= 1:
                sd[j - 1].wait()
            gd[j + 1] = pltpu.async_copy(
                tab.at[sidx.at[j + 1]], rows2[(j + 1) % 2], sg[(j + 1) % 2])
        gd[j].wait()
        sd[j] = pltpu.async_copy(rows2[j % 2], acc.at[didx.at[j]],
                                 ss[j % 2], add=True)
    sd[IDXB - 2].wait()
    sd[IDXB - 1].wait()


@functools.partial(
    pl.kernel,
    mesh=_mesh,
    compiler_params=_sc_params,
    out_type=jax.ShapeDtypeStruct((NC, NPAD, CW), jnp.float32),
    scratch_types=[
        pltpu.VMEM((IDXB, 128), jnp.int32),
        pltpu.VMEM((IDXB, 128), jnp.int32),
        pltpu.VMEM((128, CW), jnp.float32),
        pltpu.VMEM((128, CW), jnp.float32),
        pltpu.VMEM_SHARED((NPAD, CW), jnp.float32),
        pltpu.SemaphoreType.DMA,
        pltpu.SemaphoreType.DMA,
        pltpu.SemaphoreType.DMA,
        pltpu.SemaphoreType.DMA,
    ],
)
def _sc_prop16(src_hbm, dst_hbm, tab_hbm, zero_hbm, out_hbm,
               sidx, didx, rows_a, rows_b, acc, sg0, sg1, ss0, ss1):
    cid = lax.axis_index("c")
    sid = lax.axis_index("s")
    pltpu.sync_copy(zero_hbm.at[pl.ds(sid * RPT, RPT)],
                    acc.at[pl.ds(sid * RPT, RPT)])
    plsc.subcore_barrier()
    base = (cid * NS + sid) * (ER // (NC * NS))

    @pl.loop(0, ER // (NC * NS) // IDXB)
    def _(g):
        pltpu.sync_copy(src_hbm.at[pl.ds(base + g * IDXB, IDXB)], sidx)
        pltpu.sync_copy(dst_hbm.at[pl.ds(base + g * IDXB, IDXB)], didx)
        _edge_block(tab_hbm, acc, sidx, didx, (rows_a, rows_b),
                    (sg0, sg1), (ss0, ss1))

    plsc.subcore_barrier()
    pltpu.sync_copy(
        acc.at[pl.ds(sid * RPT, RPT)], out_hbm.at[cid, pl.ds(sid * RPT, RPT)]
    )


@functools.partial(
    pl.kernel,
    mesh=_mesh,
    compiler_params=_sc_params,
    out_type=jax.ShapeDtypeStruct((4, NPAD, CW), jnp.float32),
    scratch_types=[
        pltpu.VMEM((IDXB, 128), jnp.int32),
        pltpu.VMEM((IDXB, 128), jnp.int32),
        pltpu.VMEM((128, CW), jnp.float32),
        pltpu.VMEM((128, CW), jnp.float32),
        pltpu.VMEM_SHARED((NPAD, CW), jnp.float32),
        pltpu.SemaphoreType.DMA,
        pltpu.SemaphoreType.DMA,
        pltpu.SemaphoreType.DMA,
        pltpu.SemaphoreType.DMA,
    ],
)
def _sc_prop64(src_hbm, dst_hbm, tab_hbm, zero_hbm, out_hbm,
               sidx, didx, rows_a, rows_b, acc, sg0, sg1, ss0, ss1):
    cid = lax.axis_index("c")
    sid = lax.axis_index("s")

    def chunk_body(tab, outk):
        pltpu.sync_copy(zero_hbm.at[pl.ds(sid * RPT, RPT)],
                        acc.at[pl.ds(sid * RPT, RPT)])
        plsc.subcore_barrier()
        base = sid * (ER // NS)

        @pl.loop(0, ER // NS // IDXB)
        def _(g):
            pltpu.sync_copy(src_hbm.at[pl.ds(base + g * IDXB, IDXB)], sidx)
            pltpu.sync_copy(dst_hbm.at[pl.ds(base + g * IDXB, IDXB)], didx)
            _edge_block(tab, acc, sidx, didx, (rows_a, rows_b),
                        (sg0, sg1), (ss0, ss1))

        plsc.subcore_barrier()
        pltpu.sync_copy(acc.at[pl.ds(sid * RPT, RPT)],
                        outk.at[pl.ds(sid * RPT, RPT)])
        plsc.subcore_barrier()

    @pl.when(cid == 0)
    def _():
        chunk_body(tab_hbm.at[0], out_hbm.at[0])
        chunk_body(tab_hbm.at[1], out_hbm.at[1])

    @pl.when(cid == 1)
    def _():
        chunk_body(tab_hbm.at[2], out_hbm.at[2])
        chunk_body(tab_hbm.at[3], out_hbm.at[3])


BL1 = 1024
BR = BL1 * CW // 128

def _tc1_body(degp_ref, x_ref, dinv_ref, g1_ref):
    deg = degp_ref[0] + degp_ref[1] + 1.0
    dv = lax.rsqrt(deg)
    dinv_ref[...] = dv
    g1_ref[...] = x_ref[...] * dv


def _tc2_body(s1p_ref, g1_ref, dinv_ref, w1_ref, b1_ref, w2_ref, g2_ref):
    dv = dinv_ref[...]
    z128 = (s1p_ref[0] + s1p_ref[1] + g1_ref[...]) * dv
    g2a = []
    for a in range(8):
        z_a = z128[:, a * CW:(a + 1) * CW]
        h_a = jnp.dot(z_a, w1_ref[...], preferred_element_type=jnp.float32,
                      precision=lax.Precision.HIGHEST)
        h_a = jnp.maximum(h_a + b1_ref[...], 0.0)
        q_a = jnp.dot(h_a, w2_ref[...], preferred_element_type=jnp.float32,
                      precision=lax.Precision.HIGHEST)
        g2a.append(q_a * dv[:, a * CW:a * CW + 1])
    for c in range(4):
        g2_ref[c] = jnp.concatenate(
            [g2a[b][:, c * CW:(c + 1) * CW] for b in range(8)], axis=1)


def _tc3_body(s2_ref, g2_ref, dinv_ref, b2_ref, out_ref):
    dv = dinv_ref[...]
    for c in range(4):
        out_ref[c] = (s2_ref[c] + g2_ref[c]) * dv + b2_ref[c]


def kernel(x, edge_index, W1, b1, W2, b2):
    src = edge_index[0]
    dst = edge_index[1]
    pad = EPAD - N_EDGES
    pad_idx = N_NODES + (jnp.arange(pad, dtype=jnp.int32) % (NPAD - N_NODES))
    src2d = jnp.concatenate([src, pad_idx]).reshape(ER, 128)
    dst2d = jnp.concatenate([dst, pad_idx]).reshape(ER, 128)
    LIN = (NPAD * CW // 128, 128)
    x16 = jnp.pad(x, ((0, NPAD - N_NODES), (0, CW - IN_C))).reshape(LIN)
    w1_pad = jnp.pad(W1, ((0, CW - IN_C), (0, 0)))
    b1r = b1.reshape(1, HID_C)
    b2r = b2.reshape(1, OUT_C)
    zeros_n = jnp.zeros((NPAD, CW), jnp.float32)

    degp = _sc_degree(dst2d, zeros_n).reshape(NC, *LIN)

    dinv, g1 = pl.pallas_call(
        _tc1_body,
        grid=(NPAD // BL1,),
        in_specs=[
            pl.BlockSpec((NC, BR, 128), lambda i: (0, i, 0)),
            pl.BlockSpec((BR, 128), lambda i: (i, 0)),
        ],
        out_specs=[
            pl.BlockSpec((BR, 128), lambda i: (i, 0)),
            pl.BlockSpec((BR, 128), lambda i: (i, 0)),
        ],
        out_shape=[
            jax.ShapeDtypeStruct(LIN, jnp.float32),
            jax.ShapeDtypeStruct(LIN, jnp.float32),
        ],
    )(degp, x16)

    s1p = _sc_prop16(src2d, dst2d, g1.reshape(NPAD, CW),
                     zeros_n).reshape(NC, *LIN)

    g2 = pl.pallas_call(
        _tc2_body,
        grid=(NPAD // BL1,),
        in_specs=[
            pl.BlockSpec((NC, BR, 128), lambda i: (0, i, 0)),
            pl.BlockSpec((BR, 128), lambda i: (i, 0)),
            pl.BlockSpec((BR, 128), lambda i: (i, 0)),
            pl.BlockSpec((CW, HID_C), lambda i: (0, 0)),
            pl.BlockSpec((1, HID_C), lambda i: (0, 0)),
            pl.BlockSpec((HID_C, OUT_C), lambda i: (0, 0)),
        ],
        out_specs=pl.BlockSpec((4, BR, 128), lambda i: (0, i, 0)),
        out_shape=jax.ShapeDtypeStruct((4, *LIN), jnp.float32),
    )(s1p, g1, dinv, w1_pad, b1r, W2)

    s2 = _sc_prop64(src2d, dst2d, g2.reshape(4, NPAD, CW),
                    zeros_n).reshape(4, *LIN)

    b2lin = jnp.tile(b2.reshape(4, CW), (1, 8)).reshape(4, 1, 128)

    t4 = pl.pallas_call(
        _tc3_body,
        grid=(NPAD // BL1,),
        in_specs=[
            pl.BlockSpec((4, BR, 128), lambda i: (0, i, 0)),
            pl.BlockSpec((4, BR, 128), lambda i: (0, i, 0)),
            pl.BlockSpec((BR, 128), lambda i: (i, 0)),
            pl.BlockSpec((4, 1, 128), lambda i: (0, 0, 0)),
        ],
        out_specs=pl.BlockSpec((4, BR, 128), lambda i: (0, i, 0)),
        out_shape=jax.ShapeDtypeStruct((4, *LIN), jnp.float32),
    )(s2, g2, dinv, b2lin)

    t4 = t4.reshape(4, NPAD, CW)
    out_t = jnp.concatenate([t4[c].T for c in range(4)], axis=0)
    return out_t.T[:N_NODES]


# speedup vs baseline: 29.1475x; 1.3920x over previous
import functools

import jax
import jax.numpy as jnp
from jax import lax
from jax.experimental import pallas as pl
from jax.experimental.pallas import tpu as pltpu
from jax.experimental.pallas import tpu_sc as plsc

N_NODES = 100000
N_EDGES = 1600000
IN_C, HID_C, OUT_C = 12, 128, 64

NC, NS = 2, 16
CW = 16
NPAD = 100352
RPT = NPAD // NS
ER = 12800
EPAD = ER * 128
IDXB = 8

_mesh = plsc.VectorSubcoreMesh(core_axis_name="c", subcore_axis_name="s")
_sc_params = pltpu.CompilerParams(use_tc_tiling_on_sc=False)


@functools.partial(
    pl.kernel,
    mesh=_mesh,
    compiler_params=_sc_params,
    out_type=jax.ShapeDtypeStruct((NC, NPAD, CW), jnp.float32),
    scratch_types=[
        pltpu.VMEM((IDXB, 128), jnp.int32),
        pltpu.VMEM((128, CW), jnp.float32),
        pltpu.VMEM_SHARED((NPAD, CW), jnp.float32),
        pltpu.SemaphoreType.DMA,
    ],
)
def _sc_degree(dst_hbm, zero_hbm, out_hbm, didx, ones_v, acc, sem):
    cid = lax.axis_index("c")
    sid = lax.axis_index("s")

    @pl.loop(0, 128)
    def _(i):
        ones_v[i, :] = jnp.full((CW,), 1.0, jnp.float32)

    pltpu.sync_copy(zero_hbm.at[pl.ds(sid * RPT, RPT)],
                    acc.at[pl.ds(sid * RPT, RPT)])
    plsc.subcore_barrier()
    base = (cid * NS + sid) * (ER // (NC * NS))

    @pl.loop(0, ER // (NC * NS) // IDXB)
    def _(g):
        pltpu.sync_copy(dst_hbm.at[pl.ds(base + g * IDXB, IDXB)], didx)
        descs = [pltpu.async_copy(ones_v, acc.at[didx.at[j]], sem, add=True)
                 for j in range(IDXB)]
        for d in descs:
            d.wait()

    plsc.subcore_barrier()
    pltpu.sync_copy(
        acc.at[pl.ds(sid * RPT, RPT)], out_hbm.at[cid, pl.ds(sid * RPT, RPT)]
    )


def _edge_block(tab, acc, sidx, didx, rows4, sg4, ss4):
    gd = [None] * IDXB
    sd = [None] * IDXB
    for k in range(3):
        gd[k] = pltpu.async_copy(tab.at[sidx.at[k]], rows4[k], sg4[k])
    for j in range(IDXB):
        t = j + 3
        if t < IDXB:
            if j >= 1:
                sd[j - 1].wait()
            gd[t] = pltpu.async_copy(tab.at[sidx.at[t]], rows4[t % 4],
                                     sg4[t % 4])
        gd[j].wait()
        sd[j] = pltpu.async_copy(rows4[j % 4], acc.at[didx.at[j]],
                                 ss4[j % 4], add=True)
    for j in range(IDXB - 4, IDXB):
        sd[j].wait()


def _issue_idx(src_hbm, dst_hbm, row0, sidx, didx, semi):
    pltpu.async_copy(src_hbm.at[pl.ds(row0, IDXB)], sidx, semi)
    pltpu.async_copy(dst_hbm.at[pl.ds(row0, IDXB)], didx, semi)


def _wait_idx(src_hbm, dst_hbm, sidx, didx, semi):
    pltpu.make_async_copy(src_hbm.at[pl.ds(0, IDXB)], sidx, semi).wait()
    pltpu.make_async_copy(dst_hbm.at[pl.ds(0, IDXB)], didx, semi).wait()


def _pipelined_walk(src_hbm, dst_hbm, tab, acc, base, nblocks, bufs):
    (sidx_a, didx_a, sidx_b, didx_b, rows4, si_a, si_b, sg4, ss4) = bufs
    _issue_idx(src_hbm, dst_hbm, base, sidx_a, didx_a, si_a)

    @pl.loop(0, nblocks // 2)
    def _(h):
        g0 = 2 * h
        _wait_idx(src_hbm, dst_hbm, sidx_a, didx_a, si_a)
        _issue_idx(src_hbm, dst_hbm, base + (g0 + 1) * IDXB,
                   sidx_b, didx_b, si_b)
        _edge_block(tab, acc, sidx_a, didx_a, rows4, sg4, ss4)
        _wait_idx(src_hbm, dst_hbm, sidx_b, didx_b, si_b)
        nxt = lax.rem(g0 + 2, nblocks)
        _issue_idx(src_hbm, dst_hbm, base + nxt * IDXB, sidx_a, didx_a, si_a)
        _edge_block(tab, acc, sidx_b, didx_b, rows4, sg4, ss4)

    _wait_idx(src_hbm, dst_hbm, sidx_a, didx_a, si_a)


@functools.partial(
    pl.kernel,
    mesh=_mesh,
    compiler_params=_sc_params,
    out_type=jax.ShapeDtypeStruct((NC, NPAD, CW), jnp.float32),
    scratch_types=[
        pltpu.VMEM((IDXB, 128), jnp.int32),
        pltpu.VMEM((IDXB, 128), jnp.int32),
        pltpu.VMEM((IDXB, 128), jnp.int32),
        pltpu.VMEM((IDXB, 128), jnp.int32),
        pltpu.VMEM((128, CW), jnp.float32),
        pltpu.VMEM((128, CW), jnp.float32),
        pltpu.VMEM((128, CW), jnp.float32),
        pltpu.VMEM((128, CW), jnp.float32),
        pltpu.VMEM_SHARED((NPAD, CW), jnp.float32),
        pltpu.SemaphoreType.DMA,
        pltpu.SemaphoreType.DMA,
        pltpu.SemaphoreType.DMA,
        pltpu.SemaphoreType.DMA,
        pltpu.SemaphoreType.DMA,
        pltpu.SemaphoreType.DMA,
        pltpu.SemaphoreType.DMA,
        pltpu.SemaphoreType.DMA,
        pltpu.SemaphoreType.DMA,
        pltpu.SemaphoreType.DMA,
    ],
)
def _sc_prop16(src_hbm, dst_hbm, tab_hbm, zero_hbm, out_hbm,
               sidx_a, didx_a, sidx_b, didx_b, r0, r1, r2, r3, acc,
               si_a, si_b, sg0, sg1, sg2, sg3, ss0, ss1, ss2, ss3):
    cid = lax.axis_index("c")
    sid = lax.axis_index("s")
    pltpu.sync_copy(zero_hbm.at[pl.ds(sid * RPT, RPT)],
                    acc.at[pl.ds(sid * RPT, RPT)])
    plsc.subcore_barrier()
    base = (cid * NS + sid) * (ER // (NC * NS))
    bufs = (sidx_a, didx_a, sidx_b, didx_b, (r0, r1, r2, r3),
            si_a, si_b, (sg0, sg1, sg2, sg3), (ss0, ss1, ss2, ss3))
    _pipelined_walk(src_hbm, dst_hbm, tab_hbm, acc, base,
                    ER // (NC * NS) // IDXB, bufs)
    plsc.subcore_barrier()
    pltpu.sync_copy(
        acc.at[pl.ds(sid * RPT, RPT)], out_hbm.at[cid, pl.ds(sid * RPT, RPT)]
    )


@functools.partial(
    pl.kernel,
    mesh=_mesh,
    compiler_params=_sc_params,
    out_type=jax.ShapeDtypeStruct((4, NPAD, CW), jnp.float32),
    scratch_types=[
        pltpu.VMEM((IDXB, 128), jnp.int32),
        pltpu.VMEM((IDXB, 128), jnp.int32),
        pltpu.VMEM((IDXB, 128), jnp.int32),
        pltpu.VMEM((IDXB, 128), jnp.int32),
        pltpu.VMEM((128, CW), jnp.float32),
        pltpu.VMEM((128, CW), jnp.float32),
        pltpu.VMEM((128, CW), jnp.float32),
        pltpu.VMEM((128, CW), jnp.float32),
        pltpu.VMEM_SHARED((NPAD, CW), jnp.float32),
        pltpu.SemaphoreType.DMA,
        pltpu.SemaphoreType.DMA,
        pltpu.SemaphoreType.DMA,
        pltpu.SemaphoreType.DMA,
        pltpu.SemaphoreType.DMA,
        pltpu.SemaphoreType.DMA,
        pltpu.SemaphoreType.DMA,
        pltpu.SemaphoreType.DMA,
        pltpu.SemaphoreType.DMA,
        pltpu.SemaphoreType.DMA,
    ],
)
def _sc_prop64(src_hbm, dst_hbm, tab_hbm, zero_hbm, out_hbm,
               sidx_a, didx_a, sidx_b, didx_b, r0, r1, r2, r3, acc,
               si_a, si_b, sg0, sg1, sg2, sg3, ss0, ss1, ss2, ss3):
    cid = lax.axis_index("c")
    sid = lax.axis_index("s")
    bufs = (sidx_a, didx_a, sidx_b, didx_b, (r0, r1, r2, r3),
            si_a, si_b, (sg0, sg1, sg2, sg3), (ss0, ss1, ss2, ss3))

    def chunk_body(tab, outk):
        pltpu.sync_copy(zero_hbm.at[pl.ds(sid * RPT, RPT)],
                        acc.at[pl.ds(sid * RPT, RPT)])
        plsc.subcore_barrier()
        base = sid * (ER // NS)
        _pipelined_walk(src_hbm, dst_hbm, tab, acc, base,
                        ER // NS // IDXB, bufs)
        plsc.subcore_barrier()
        pltpu.sync_copy(acc.at[pl.ds(sid * RPT, RPT)],
                        outk.at[pl.ds(sid * RPT, RPT)])
        plsc.subcore_barrier()

    @pl.when(cid == 0)
    def _():
        chunk_body(tab_hbm.at[0], out_hbm.at[0])
        chunk_body(tab_hbm.at[1], out_hbm.at[1])

    @pl.when(cid == 1)
    def _():
        chunk_body(tab_hbm.at[2], out_hbm.at[2])
        chunk_body(tab_hbm.at[3], out_hbm.at[3])


BL1 = 1024
BR = BL1 * CW // 128

def _tc1_body(degp_ref, x_ref, dinv_ref, g1_ref):
    deg = degp_ref[0] + degp_ref[1] + 1.0
    dv = lax.rsqrt(deg)
    dinv_ref[...] = dv
    g1_ref[...] = x_ref[...] * dv


def _tc2_body(s1p_ref, g1_ref, dinv_ref, w1_ref, b1_ref, w2_ref, g2_ref):
    dv = dinv_ref[...]
    z128 = (s1p_ref[0] + s1p_ref[1] + g1_ref[...]) * dv
    g2a = []
    for a in range(8):
        z_a = z128[:, a * CW:(a + 1) * CW]
        h_a = jnp.dot(z_a, w1_ref[...], preferred_element_type=jnp.float32,
                      precision=lax.Precision.HIGHEST)
        h_a = jnp.maximum(h_a + b1_ref[...], 0.0)
        q_a = jnp.dot(h_a, w2_ref[...], preferred_element_type=jnp.float32,
                      precision=lax.Precision.HIGHEST)
        g2a.append(q_a * dv[:, a * CW:a * CW + 1])
    for c in range(4):
        g2_ref[c] = jnp.concatenate(
            [g2a[b][:, c * CW:(c + 1) * CW] for b in range(8)], axis=1)


def _tc3_body(s2_ref, g2_ref, dinv_ref, b2_ref, out_ref):
    dv = dinv_ref[...]
    for c in range(4):
        out_ref[c] = (s2_ref[c] + g2_ref[c]) * dv + b2_ref[c]


def kernel(x, edge_index, W1, b1, W2, b2):
    src = edge_index[0]
    dst = edge_index[1]
    pad = EPAD - N_EDGES
    pad_idx = N_NODES + (jnp.arange(pad, dtype=jnp.int32) % (NPAD - N_NODES))
    src2d = jnp.concatenate([src, pad_idx]).reshape(ER, 128)
    dst2d = jnp.concatenate([dst, pad_idx]).reshape(ER, 128)
    LIN = (NPAD * CW // 128, 128)
    x16 = jnp.pad(x, ((0, NPAD - N_NODES), (0, CW - IN_C))).reshape(LIN)
    w1_pad = jnp.pad(W1, ((0, CW - IN_C), (0, 0)))
    b1r = b1.reshape(1, HID_C)
    b2r = b2.reshape(1, OUT_C)
    zeros_n = jnp.zeros((NPAD, CW), jnp.float32)

    degp = _sc_degree(dst2d, zeros_n).reshape(NC, *LIN)

    dinv, g1 = pl.pallas_call(
        _tc1_body,
        grid=(NPAD // BL1,),
        in_specs=[
            pl.BlockSpec((NC, BR, 128), lambda i: (0, i, 0)),
            pl.BlockSpec((BR, 128), lambda i: (i, 0)),
        ],
        out_specs=[
            pl.BlockSpec((BR, 128), lambda i: (i, 0)),
            pl.BlockSpec((BR, 128), lambda i: (i, 0)),
        ],
        out_shape=[
            jax.ShapeDtypeStruct(LIN, jnp.float32),
            jax.ShapeDtypeStruct(LIN, jnp.float32),
        ],
    )(degp, x16)

    s1p = _sc_prop16(src2d, dst2d, g1.reshape(NPAD, CW),
                     zeros_n).reshape(NC, *LIN)

    g2 = pl.pallas_call(
        _tc2_body,
        grid=(NPAD // BL1,),
        in_specs=[
            pl.BlockSpec((NC, BR, 128), lambda i: (0, i, 0)),
            pl.BlockSpec((BR, 128), lambda i: (i, 0)),
            pl.BlockSpec((BR, 128), lambda i: (i, 0)),
            pl.BlockSpec((CW, HID_C), lambda i: (0, 0)),
            pl.BlockSpec((1, HID_C), lambda i: (0, 0)),
            pl.BlockSpec((HID_C, OUT_C), lambda i: (0, 0)),
        ],
        out_specs=pl.BlockSpec((4, BR, 128), lambda i: (0, i, 0)),
        out_shape=jax.ShapeDtypeStruct((4, *LIN), jnp.float32),
    )(s1p, g1, dinv, w1_pad, b1r, W2)

    s2 = _sc_prop64(src2d, dst2d, g2.reshape(4, NPAD, CW),
                    zeros_n).reshape(4, *LIN)

    b2lin = jnp.tile(b2.reshape(4, CW), (1, 8)).reshape(4, 1, 128)

    t4 = pl.pallas_call(
        _tc3_body,
        grid=(NPAD // BL1,),
        in_specs=[
            pl.BlockSpec((4, BR, 128), lambda i: (0, i, 0)),
            pl.BlockSpec((4, BR, 128), lambda i: (0, i, 0)),
            pl.BlockSpec((BR, 128), lambda i: (i, 0)),
            pl.BlockSpec((4, 1, 128), lambda i: (0, 0, 0)),
        ],
        out_specs=pl.BlockSpec((4, BR, 128), lambda i: (0, i, 0)),
        out_shape=jax.ShapeDtypeStruct((4, *LIN), jnp.float32),
    )(s2, g2, dinv, b2lin)

    t4 = t4.reshape(4, NPAD, CW)
    out_t = jnp.concatenate([t4[c].T for c in range(4)], axis=0)
    return out_t.T[:N_NODES]


# speedup vs baseline: 32.4369x; 1.1129x over previous
import functools

import jax
import jax.numpy as jnp
from jax import lax
from jax.experimental import pallas as pl
from jax.experimental.pallas import tpu as pltpu
from jax.experimental.pallas import tpu_sc as plsc

N_NODES = 100000
N_EDGES = 1600000
IN_C, HID_C, OUT_C = 12, 128, 64

NC, NS = 2, 16
CW = 16
NPAD = 100352
RPT = NPAD // NS
ER = 12800
EPAD = ER * 128
IDXB = 8

_mesh = plsc.VectorSubcoreMesh(core_axis_name="c", subcore_axis_name="s")
_sc_params = pltpu.CompilerParams(use_tc_tiling_on_sc=False)


@functools.partial(
    pl.kernel,
    mesh=_mesh,
    compiler_params=_sc_params,
    out_type=jax.ShapeDtypeStruct((NC, NPAD, CW), jnp.float32),
    scratch_types=[
        pltpu.VMEM((IDXB, 128), jnp.int32),
        pltpu.VMEM((128, CW), jnp.float32),
        pltpu.VMEM_SHARED((NPAD, CW), jnp.float32),
        pltpu.SemaphoreType.DMA,
    ],
)
def _sc_degree(dst_hbm, zero_hbm, out_hbm, didx, ones_v, acc, sem):
    cid = lax.axis_index("c")
    sid = lax.axis_index("s")

    @pl.loop(0, 128)
    def _(i):
        ones_v[i, :] = jnp.full((CW,), 1.0, jnp.float32)

    pltpu.sync_copy(zero_hbm.at[pl.ds(sid * RPT, RPT)],
                    acc.at[pl.ds(sid * RPT, RPT)])
    plsc.subcore_barrier()
    base = (cid * NS + sid) * (ER // (NC * NS))

    @pl.loop(0, ER // (NC * NS) // IDXB)
    def _(g):
        pltpu.sync_copy(dst_hbm.at[pl.ds(base + g * IDXB, IDXB)], didx)
        descs = [pltpu.async_copy(ones_v, acc.at[didx.at[j]], sem, add=True)
                 for j in range(IDXB)]
        for d in descs:
            d.wait()

    plsc.subcore_barrier()
    pltpu.sync_copy(
        acc.at[pl.ds(sid * RPT, RPT)], out_hbm.at[cid, pl.ds(sid * RPT, RPT)]
    )


def _edge_block(tab, acc, sidx, didx, rows4, sg4, ss4):
    gd = [None] * IDXB
    sd = [None] * IDXB
    for k in range(3):
        gd[k] = pltpu.async_copy(tab.at[sidx.at[k]], rows4[k], sg4[k])
    for j in range(IDXB):
        t = j + 3
        if t < IDXB:
            if j >= 1:
                sd[j - 1].wait()
            gd[t] = pltpu.async_copy(tab.at[sidx.at[t]], rows4[t % 4],
                                     sg4[t % 4])
        gd[j].wait()
        sd[j] = pltpu.async_copy(rows4[j % 4], acc.at[didx.at[j]],
                                 ss4[j % 4], add=True)
    for j in range(IDXB - 4, IDXB):
        sd[j].wait()


def _issue_idx(src_hbm, dst_hbm, row0, sidx, didx, semi):
    pltpu.async_copy(src_hbm.at[pl.ds(row0, IDXB)], sidx, semi)
    pltpu.async_copy(dst_hbm.at[pl.ds(row0, IDXB)], didx, semi)


def _wait_idx(src_hbm, dst_hbm, sidx, didx, semi):
    pltpu.make_async_copy(src_hbm.at[pl.ds(0, IDXB)], sidx, semi).wait()
    pltpu.make_async_copy(dst_hbm.at[pl.ds(0, IDXB)], didx, semi).wait()


def _pipelined_walk(src_hbm, dst_hbm, tab, acc, base, nblocks, bufs):
    (sidx_a, didx_a, sidx_b, didx_b, rows4, si_a, si_b, sg4, ss4) = bufs
    _issue_idx(src_hbm, dst_hbm, base, sidx_a, didx_a, si_a)

    @pl.loop(0, nblocks // 2)
    def _(h):
        g0 = 2 * h
        _wait_idx(src_hbm, dst_hbm, sidx_a, didx_a, si_a)
        _issue_idx(src_hbm, dst_hbm, base + (g0 + 1) * IDXB,
                   sidx_b, didx_b, si_b)
        _edge_block(tab, acc, sidx_a, didx_a, rows4, sg4, ss4)
        _wait_idx(src_hbm, dst_hbm, sidx_b, didx_b, si_b)
        nxt = lax.rem(g0 + 2, nblocks)
        _issue_idx(src_hbm, dst_hbm, base + nxt * IDXB, sidx_a, didx_a, si_a)
        _edge_block(tab, acc, sidx_b, didx_b, rows4, sg4, ss4)

    _wait_idx(src_hbm, dst_hbm, sidx_a, didx_a, si_a)


@functools.partial(
    pl.kernel,
    mesh=_mesh,
    compiler_params=_sc_params,
    out_type=jax.ShapeDtypeStruct((NC, NPAD, CW), jnp.float32),
    scratch_types=[
        pltpu.VMEM((IDXB, 128), jnp.int32),
        pltpu.VMEM((IDXB, 128), jnp.int32),
        pltpu.VMEM((IDXB, 128), jnp.int32),
        pltpu.VMEM((IDXB, 128), jnp.int32),
        pltpu.VMEM((128, CW), jnp.float32),
        pltpu.VMEM((128, CW), jnp.float32),
        pltpu.VMEM((128, CW), jnp.float32),
        pltpu.VMEM((128, CW), jnp.float32),
        pltpu.VMEM_SHARED((NPAD, CW), jnp.float32),
        pltpu.SemaphoreType.DMA,
        pltpu.SemaphoreType.DMA,
        pltpu.SemaphoreType.DMA,
        pltpu.SemaphoreType.DMA,
        pltpu.SemaphoreType.DMA,
        pltpu.SemaphoreType.DMA,
        pltpu.SemaphoreType.DMA,
        pltpu.SemaphoreType.DMA,
        pltpu.SemaphoreType.DMA,
        pltpu.SemaphoreType.DMA,
    ],
)
def _sc_prop16(src_hbm, dst_hbm, tab_hbm, zero_hbm, out_hbm,
               sidx_a, didx_a, sidx_b, didx_b, r0, r1, r2, r3, acc,
               si_a, si_b, sg0, sg1, sg2, sg3, ss0, ss1, ss2, ss3):
    cid = lax.axis_index("c")
    sid = lax.axis_index("s")
    pltpu.sync_copy(zero_hbm.at[pl.ds(sid * RPT, RPT)],
                    acc.at[pl.ds(sid * RPT, RPT)])
    plsc.subcore_barrier()
    base = (cid * NS + sid) * (ER // (NC * NS))
    bufs = (sidx_a, didx_a, sidx_b, didx_b, (r0, r1, r2, r3),
            si_a, si_b, (sg0, sg1, sg2, sg3), (ss0, ss1, ss2, ss3))
    _pipelined_walk(src_hbm, dst_hbm, tab_hbm, acc, base,
                    ER // (NC * NS) // IDXB, bufs)
    plsc.subcore_barrier()
    pltpu.sync_copy(
        acc.at[pl.ds(sid * RPT, RPT)], out_hbm.at[cid, pl.ds(sid * RPT, RPT)]
    )


@functools.partial(
    pl.kernel,
    mesh=_mesh,
    compiler_params=_sc_params,
    out_type=jax.ShapeDtypeStruct((4, NPAD, CW), jnp.float32),
    scratch_types=[
        pltpu.VMEM((IDXB, 128), jnp.int32),
        pltpu.VMEM((IDXB, 128), jnp.int32),
        pltpu.VMEM((IDXB, 128), jnp.int32),
        pltpu.VMEM((IDXB, 128), jnp.int32),
        pltpu.VMEM((128, CW), jnp.float32),
        pltpu.VMEM((128, CW), jnp.float32),
        pltpu.VMEM((128, CW), jnp.float32),
        pltpu.VMEM((128, CW), jnp.float32),
        pltpu.VMEM_SHARED((NPAD, CW), jnp.float32),
        pltpu.SemaphoreType.DMA,
        pltpu.SemaphoreType.DMA,
        pltpu.SemaphoreType.DMA,
        pltpu.SemaphoreType.DMA,
        pltpu.SemaphoreType.DMA,
        pltpu.SemaphoreType.DMA,
        pltpu.SemaphoreType.DMA,
        pltpu.SemaphoreType.DMA,
        pltpu.SemaphoreType.DMA,
        pltpu.SemaphoreType.DMA,
    ],
)
def _sc_prop64(src_hbm, dst_hbm, tab_hbm, zero_hbm, out_hbm,
               sidx_a, didx_a, sidx_b, didx_b, r0, r1, r2, r3, acc,
               si_a, si_b, sg0, sg1, sg2, sg3, ss0, ss1, ss2, ss3):
    cid = lax.axis_index("c")
    sid = lax.axis_index("s")
    bufs = (sidx_a, didx_a, sidx_b, didx_b, (r0, r1, r2, r3),
            si_a, si_b, (sg0, sg1, sg2, sg3), (ss0, ss1, ss2, ss3))

    def chunk_body(tab, outk):
        pltpu.sync_copy(zero_hbm.at[pl.ds(sid * RPT, RPT)],
                        acc.at[pl.ds(sid * RPT, RPT)])
        plsc.subcore_barrier()
        base = sid * (ER // NS)
        _pipelined_walk(src_hbm, dst_hbm, tab, acc, base,
                        ER // NS // IDXB, bufs)
        plsc.subcore_barrier()
        pltpu.sync_copy(acc.at[pl.ds(sid * RPT, RPT)],
                        outk.at[pl.ds(sid * RPT, RPT)])
        plsc.subcore_barrier()

    @pl.when(cid == 0)
    def _():
        chunk_body(tab_hbm.at[0], out_hbm.at[0])
        chunk_body(tab_hbm.at[1], out_hbm.at[1])

    @pl.when(cid == 1)
    def _():
        chunk_body(tab_hbm.at[2], out_hbm.at[2])
        chunk_body(tab_hbm.at[3], out_hbm.at[3])


BL1 = 1024
BR = BL1 * CW // 128

def _tc1_body(degp_ref, x_ref, dinv_ref, g1_ref):
    deg = degp_ref[0] + degp_ref[1] + 1.0
    dv = lax.rsqrt(deg)
    dinv_ref[...] = dv
    g1_ref[...] = x_ref[...] * dv


def _tc2_body(s1p_ref, g1_ref, dinv_ref, w1_ref, b1_ref, w2_ref, g2_ref):
    dv = dinv_ref[...]
    z128 = (s1p_ref[0] + s1p_ref[1] + g1_ref[...]) * dv
    g2a = []
    for a in range(8):
        z_a = z128[:, a * CW:(a + 1) * CW]
        h_a = jnp.dot(z_a, w1_ref[...], preferred_element_type=jnp.float32)
        h_a = jnp.maximum(h_a + b1_ref[...], 0.0)
        q_a = jnp.dot(h_a, w2_ref[...], preferred_element_type=jnp.float32)
        g2a.append(q_a * dv[:, a * CW:a * CW + 1])
    for c in range(4):
        g2_ref[c] = jnp.concatenate(
            [g2a[b][:, c * CW:(c + 1) * CW] for b in range(8)], axis=1)


def _tc3_body(s2_ref, g2_ref, dinv_ref, b2_ref, out_ref):
    dv = dinv_ref[...]
    for c in range(4):
        out_ref[c] = (s2_ref[c] + g2_ref[c]) * dv + b2_ref[c]


def kernel(x, edge_index, W1, b1, W2, b2):
    src = edge_index[0]
    dst = edge_index[1]
    pad = EPAD - N_EDGES
    pad_idx = N_NODES + (jnp.arange(pad, dtype=jnp.int32) % (NPAD - N_NODES))
    src2d = jnp.concatenate([src, pad_idx]).reshape(ER, 128)
    dst2d = jnp.concatenate([dst, pad_idx]).reshape(ER, 128)
    LIN = (NPAD * CW // 128, 128)
    x16 = jnp.pad(x, ((0, NPAD - N_NODES), (0, CW - IN_C))).reshape(LIN)
    w1_pad = jnp.pad(W1, ((0, CW - IN_C), (0, 0)))
    b1r = b1.reshape(1, HID_C)
    b2r = b2.reshape(1, OUT_C)
    zeros_n = jnp.zeros((NPAD, CW), jnp.float32)

    degp = _sc_degree(dst2d, zeros_n).reshape(NC, *LIN)

    dinv, g1 = pl.pallas_call(
        _tc1_body,
        grid=(NPAD // BL1,),
        in_specs=[
            pl.BlockSpec((NC, BR, 128), lambda i: (0, i, 0)),
            pl.BlockSpec((BR, 128), lambda i: (i, 0)),
        ],
        out_specs=[
            pl.BlockSpec((BR, 128), lambda i: (i, 0)),
            pl.BlockSpec((BR, 128), lambda i: (i, 0)),
        ],
        out_shape=[
            jax.ShapeDtypeStruct(LIN, jnp.float32),
            jax.ShapeDtypeStruct(LIN, jnp.float32),
        ],
    )(degp, x16)

    s1p = _sc_prop16(src2d, dst2d, g1.reshape(NPAD, CW),
                     zeros_n).reshape(NC, *LIN)

    g2 = pl.pallas_call(
        _tc2_body,
        grid=(NPAD // BL1,),
        in_specs=[
            pl.BlockSpec((NC, BR, 128), lambda i: (0, i, 0)),
            pl.BlockSpec((BR, 128), lambda i: (i, 0)),
            pl.BlockSpec((BR, 128), lambda i: (i, 0)),
            pl.BlockSpec((CW, HID_C), lambda i: (0, 0)),
            pl.BlockSpec((1, HID_C), lambda i: (0, 0)),
            pl.BlockSpec((HID_C, OUT_C), lambda i: (0, 0)),
        ],
        out_specs=pl.BlockSpec((4, BR, 128), lambda i: (0, i, 0)),
        out_shape=jax.ShapeDtypeStruct((4, *LIN), jnp.float32),
    )(s1p, g1, dinv, w1_pad, b1r, W2)

    s2 = _sc_prop64(src2d, dst2d, g2.reshape(4, NPAD, CW),
                    zeros_n).reshape(4, *LIN)

    b2lin = jnp.tile(b2.reshape(4, CW), (1, 8)).reshape(4, 1, 128)

    t4 = pl.pallas_call(
        _tc3_body,
        grid=(NPAD // BL1,),
        in_specs=[
            pl.BlockSpec((4, BR, 128), lambda i: (0, i, 0)),
            pl.BlockSpec((4, BR, 128), lambda i: (0, i, 0)),
            pl.BlockSpec((BR, 128), lambda i: (i, 0)),
            pl.BlockSpec((4, 1, 128), lambda i: (0, 0, 0)),
        ],
        out_specs=pl.BlockSpec((4, BR, 128), lambda i: (0, i, 0)),
        out_shape=jax.ShapeDtypeStruct((4, *LIN), jnp.float32),
    )(s2, g2, dinv, b2lin)

    out_t = t4.reshape(4, NPAD // 8, 8, CW).transpose(0, 3, 1, 2)
    out_t = out_t.reshape(4 * CW, NPAD)
    return out_t[:, :N_NODES].T


# speedup vs baseline: 33.1959x; 1.0234x over previous
import functools

import jax
import jax.numpy as jnp
from jax import lax
from jax.experimental import pallas as pl
from jax.experimental.pallas import tpu as pltpu
from jax.experimental.pallas import tpu_sc as plsc

N_NODES = 100000
N_EDGES = 1600000
IN_C, HID_C, OUT_C = 12, 128, 64

NC, NS = 2, 16
CW = 16
NPAD = 100352
RPT = NPAD // NS
ER = 12800
EPAD = ER * 128
IDXB = 8

_mesh = plsc.VectorSubcoreMesh(core_axis_name="c", subcore_axis_name="s")
_sc_params = pltpu.CompilerParams(use_tc_tiling_on_sc=False)


@functools.partial(
    pl.kernel,
    mesh=_mesh,
    compiler_params=_sc_params,
    out_type=jax.ShapeDtypeStruct((NC, NPAD, CW), jnp.float32),
    scratch_types=[
        pltpu.VMEM((IDXB, 128), jnp.int32),
        pltpu.VMEM((IDXB, 128), jnp.int32),
        pltpu.VMEM((128, CW), jnp.float32),
        pltpu.VMEM_SHARED((NPAD, CW), jnp.float32),
        pltpu.SemaphoreType.DMA,
        pltpu.SemaphoreType.DMA,
        pltpu.SemaphoreType.DMA,
    ],
)
def _sc_degree(dst_hbm, zero_hbm, out_hbm, didx_a, didx_b, ones_v, acc,
               si_a, si_b, sem):
    cid = lax.axis_index("c")
    sid = lax.axis_index("s")

    @pl.loop(0, 128)
    def _(i):
        ones_v[i, :] = jnp.full((CW,), 1.0, jnp.float32)

    pltpu.sync_copy(zero_hbm.at[pl.ds(sid * RPT, RPT)],
                    acc.at[pl.ds(sid * RPT, RPT)])
    plsc.subcore_barrier()
    nblocks = ER // (NC * NS) // IDXB
    base = (cid * NS + sid) * (ER // (NC * NS))
    pltpu.async_copy(dst_hbm.at[pl.ds(base, IDXB)], didx_a, si_a)

    def scatter_block(didx):
        descs = [pltpu.async_copy(ones_v, acc.at[didx.at[j]], sem, add=True)
                 for j in range(IDXB)]
        for d in descs:
            d.wait()

    @pl.loop(0, nblocks // 2)
    def _(h):
        g0 = 2 * h
        pltpu.make_async_copy(dst_hbm.at[pl.ds(0, IDXB)], didx_a, si_a).wait()
        pltpu.async_copy(dst_hbm.at[pl.ds(base + (g0 + 1) * IDXB, IDXB)],
                         didx_b, si_b)
        scatter_block(didx_a)
        pltpu.make_async_copy(dst_hbm.at[pl.ds(0, IDXB)], didx_b, si_b).wait()
        nxt = lax.rem(g0 + 2, nblocks)
        pltpu.async_copy(dst_hbm.at[pl.ds(base + nxt * IDXB, IDXB)],
                         didx_a, si_a)
        scatter_block(didx_b)

    pltpu.make_async_copy(dst_hbm.at[pl.ds(0, IDXB)], didx_a, si_a).wait()
    plsc.subcore_barrier()
    pltpu.sync_copy(
        acc.at[pl.ds(sid * RPT, RPT)], out_hbm.at[cid, pl.ds(sid * RPT, RPT)]
    )


def _edge_block(tab, acc, sidx, didx, rows4, sg4, ss4):
    gd = [None] * IDXB
    sd = [None] * IDXB
    for k in range(3):
        gd[k] = pltpu.async_copy(tab.at[sidx.at[k]], rows4[k], sg4[k])
    for j in range(IDXB):
        t = j + 3
        if t < IDXB:
            if j >= 1:
                sd[j - 1].wait()
            gd[t] = pltpu.async_copy(tab.at[sidx.at[t]], rows4[t % 4],
                                     sg4[t % 4])
        gd[j].wait()
        sd[j] = pltpu.async_copy(rows4[j % 4], acc.at[didx.at[j]],
                                 ss4[j % 4], add=True)
    for j in range(IDXB - 4, IDXB):
        sd[j].wait()


def _issue_idx(src_hbm, dst_hbm, row0, sidx, didx, semi):
    pltpu.async_copy(src_hbm.at[pl.ds(row0, IDXB)], sidx, semi)
    pltpu.async_copy(dst_hbm.at[pl.ds(row0, IDXB)], didx, semi)


def _wait_idx(src_hbm, dst_hbm, sidx, didx, semi):
    pltpu.make_async_copy(src_hbm.at[pl.ds(0, IDXB)], sidx, semi).wait()
    pltpu.make_async_copy(dst_hbm.at[pl.ds(0, IDXB)], didx, semi).wait()


def _pipelined_walk(src_hbm, dst_hbm, tab, acc, base, nblocks, bufs):
    (sidx_a, didx_a, sidx_b, didx_b, rows4, si_a, si_b, sg4, ss4) = bufs
    _issue_idx(src_hbm, dst_hbm, base, sidx_a, didx_a, si_a)

    @pl.loop(0, nblocks // 2)
    def _(h):
        g0 = 2 * h
        _wait_idx(src_hbm, dst_hbm, sidx_a, didx_a, si_a)
        _issue_idx(src_hbm, dst_hbm, base + (g0 + 1) * IDXB,
                   sidx_b, didx_b, si_b)
        _edge_block(tab, acc, sidx_a, didx_a, rows4, sg4, ss4)
        _wait_idx(src_hbm, dst_hbm, sidx_b, didx_b, si_b)
        nxt = lax.rem(g0 + 2, nblocks)
        _issue_idx(src_hbm, dst_hbm, base + nxt * IDXB, sidx_a, didx_a, si_a)
        _edge_block(tab, acc, sidx_b, didx_b, rows4, sg4, ss4)

    _wait_idx(src_hbm, dst_hbm, sidx_a, didx_a, si_a)


@functools.partial(
    pl.kernel,
    mesh=_mesh,
    compiler_params=_sc_params,
    out_type=jax.ShapeDtypeStruct((NC, NPAD, CW), jnp.float32),
    scratch_types=[
        pltpu.VMEM((IDXB, 128), jnp.int32),
        pltpu.VMEM((IDXB, 128), jnp.int32),
        pltpu.VMEM((IDXB, 128), jnp.int32),
        pltpu.VMEM((IDXB, 128), jnp.int32),
        pltpu.VMEM((128, CW), jnp.float32),
        pltpu.VMEM((128, CW), jnp.float32),
        pltpu.VMEM((128, CW), jnp.float32),
        pltpu.VMEM((128, CW), jnp.float32),
        pltpu.VMEM_SHARED((NPAD, CW), jnp.float32),
        pltpu.SemaphoreType.DMA,
        pltpu.SemaphoreType.DMA,
        pltpu.SemaphoreType.DMA,
        pltpu.SemaphoreType.DMA,
        pltpu.SemaphoreType.DMA,
        pltpu.SemaphoreType.DMA,
        pltpu.SemaphoreType.DMA,
        pltpu.SemaphoreType.DMA,
        pltpu.SemaphoreType.DMA,
        pltpu.SemaphoreType.DMA,
    ],
)
def _sc_prop16(src_hbm, dst_hbm, tab_hbm, zero_hbm, out_hbm,
               sidx_a, didx_a, sidx_b, didx_b, r0, r1, r2, r3, acc,
               si_a, si_b, sg0, sg1, sg2, sg3, ss0, ss1, ss2, ss3):
    cid = lax.axis_index("c")
    sid = lax.axis_index("s")
    pltpu.sync_copy(zero_hbm.at[pl.ds(sid * RPT, RPT)],
                    acc.at[pl.ds(sid * RPT, RPT)])
    plsc.subcore_barrier()
    base = (cid * NS + sid) * (ER // (NC * NS))
    bufs = (sidx_a, didx_a, sidx_b, didx_b, (r0, r1, r2, r3),
            si_a, si_b, (sg0, sg1, sg2, sg3), (ss0, ss1, ss2, ss3))
    _pipelined_walk(src_hbm, dst_hbm, tab_hbm, acc, base,
                    ER // (NC * NS) // IDXB, bufs)
    plsc.subcore_barrier()
    pltpu.sync_copy(
        acc.at[pl.ds(sid * RPT, RPT)], out_hbm.at[cid, pl.ds(sid * RPT, RPT)]
    )


@functools.partial(
    pl.kernel,
    mesh=_mesh,
    compiler_params=_sc_params,
    out_type=jax.ShapeDtypeStruct((4, NPAD, CW), jnp.float32),
    scratch_types=[
        pltpu.VMEM((IDXB, 128), jnp.int32),
        pltpu.VMEM((IDXB, 128), jnp.int32),
        pltpu.VMEM((IDXB, 128), jnp.int32),
        pltpu.VMEM((IDXB, 128), jnp.int32),
        pltpu.VMEM((128, CW), jnp.float32),
        pltpu.VMEM((128, CW), jnp.float32),
        pltpu.VMEM((128, CW), jnp.float32),
        pltpu.VMEM((128, CW), jnp.float32),
        pltpu.VMEM_SHARED((NPAD, CW), jnp.float32),
        pltpu.SemaphoreType.DMA,
        pltpu.SemaphoreType.DMA,
        pltpu.SemaphoreType.DMA,
        pltpu.SemaphoreType.DMA,
        pltpu.SemaphoreType.DMA,
        pltpu.SemaphoreType.DMA,
        pltpu.SemaphoreType.DMA,
        pltpu.SemaphoreType.DMA,
        pltpu.SemaphoreType.DMA,
        pltpu.SemaphoreType.DMA,
    ],
)
def _sc_prop64(src_hbm, dst_hbm, tab_hbm, zero_hbm, out_hbm,
               sidx_a, didx_a, sidx_b, didx_b, r0, r1, r2, r3, acc,
               si_a, si_b, sg0, sg1, sg2, sg3, ss0, ss1, ss2, ss3):
    cid = lax.axis_index("c")
    sid = lax.axis_index("s")
    bufs = (sidx_a, didx_a, sidx_b, didx_b, (r0, r1, r2, r3),
            si_a, si_b, (sg0, sg1, sg2, sg3), (ss0, ss1, ss2, ss3))

    def chunk_body(tab, outk):
        pltpu.sync_copy(zero_hbm.at[pl.ds(sid * RPT, RPT)],
                        acc.at[pl.ds(sid * RPT, RPT)])
        plsc.subcore_barrier()
        base = sid * (ER // NS)
        _pipelined_walk(src_hbm, dst_hbm, tab, acc, base,
                        ER // NS // IDXB, bufs)
        plsc.subcore_barrier()
        pltpu.sync_copy(acc.at[pl.ds(sid * RPT, RPT)],
                        outk.at[pl.ds(sid * RPT, RPT)])
        plsc.subcore_barrier()

    @pl.when(cid == 0)
    def _():
        chunk_body(tab_hbm.at[0], out_hbm.at[0])
        chunk_body(tab_hbm.at[1], out_hbm.at[1])

    @pl.when(cid == 1)
    def _():
        chunk_body(tab_hbm.at[2], out_hbm.at[2])
        chunk_body(tab_hbm.at[3], out_hbm.at[3])


BL1 = 1024
BR = BL1 * CW // 128

def _tc1_body(degp_ref, x_ref, dinv_ref, g1_ref):
    deg = degp_ref[0] + degp_ref[1] + 1.0
    dv = lax.rsqrt(deg)
    dinv_ref[...] = dv
    g1_ref[...] = x_ref[...] * dv


def _tc2_body(s1p_ref, g1_ref, dinv_ref, b1g_ref, bd1_ref, bd2_ref,
              rmat_ref, g2_ref):
    dv = dinv_ref[...]
    z128 = (s1p_ref[0] + s1p_ref[1] + g1_ref[...]) * dv
    h = jnp.dot(z128, bd1_ref[...], preferred_element_type=jnp.float32)
    h = jnp.maximum(h + b1g_ref[...], 0.0)
    q = jnp.dot(h, bd2_ref[...], preferred_element_type=jnp.float32)
    qd = q * jnp.dot(dv, rmat_ref[...], preferred_element_type=jnp.float32)
    for c in range(4):
        g2_ref[c] = jnp.concatenate(
            [qd[:, 64 * b + 16 * c:64 * b + 16 * c + CW] for b in range(8)],
            axis=1)


def _tc3_body(s2_ref, g2_ref, dinv_ref, b2_ref, out_ref):
    dv = dinv_ref[...]
    for c in range(4):
        out_ref[c] = (s2_ref[c] + g2_ref[c]) * dv + b2_ref[c]


def kernel(x, edge_index, W1, b1, W2, b2):
    src = edge_index[0]
    dst = edge_index[1]
    pad = EPAD - N_EDGES
    pad_idx = N_NODES + (jnp.arange(pad, dtype=jnp.int32) % (NPAD - N_NODES))
    src2d = jnp.concatenate([src, pad_idx]).reshape(ER, 128)
    dst2d = jnp.concatenate([dst, pad_idx]).reshape(ER, 128)
    LIN = (NPAD * CW // 128, 128)
    x16 = jnp.pad(x, ((0, NPAD - N_NODES), (0, CW - IN_C))).reshape(LIN)
    w1_pad = jnp.pad(W1, ((0, CW - IN_C), (0, 0)))
    b1r = b1.reshape(1, HID_C)
    b2r = b2.reshape(1, OUT_C)
    zeros_n = jnp.zeros((NPAD, CW), jnp.float32)

    degp = _sc_degree(dst2d, zeros_n).reshape(NC, *LIN)

    dinv, g1 = pl.pallas_call(
        _tc1_body,
        grid=(NPAD // BL1,),
        in_specs=[
            pl.BlockSpec((NC, BR, 128), lambda i: (0, i, 0)),
            pl.BlockSpec((BR, 128), lambda i: (i, 0)),
        ],
        out_specs=[
            pl.BlockSpec((BR, 128), lambda i: (i, 0)),
            pl.BlockSpec((BR, 128), lambda i: (i, 0)),
        ],
        out_shape=[
            jax.ShapeDtypeStruct(LIN, jnp.float32),
            jax.ShapeDtypeStruct(LIN, jnp.float32),
        ],
    )(degp, x16)

    s1p = _sc_prop16(src2d, dst2d, g1.reshape(NPAD, CW),
                     zeros_n).reshape(NC, *LIN)

    bd1 = jnp.kron(jnp.eye(8, dtype=jnp.float32), w1_pad)
    bd2 = jnp.kron(jnp.eye(8, dtype=jnp.float32), W2)
    b1g = jnp.tile(b1, 8).reshape(1, 8 * HID_C)
    rmat = jnp.kron(jnp.eye(8, dtype=jnp.float32),
                    jnp.full((CW, OUT_C), 1.0 / CW, jnp.float32))

    g2 = pl.pallas_call(
        _tc2_body,
        grid=(NPAD // BL1,),
        in_specs=[
            pl.BlockSpec((NC, BR, 128), lambda i: (0, i, 0)),
            pl.BlockSpec((BR, 128), lambda i: (i, 0)),
            pl.BlockSpec((BR, 128), lambda i: (i, 0)),
            pl.BlockSpec((1, 8 * HID_C), lambda i: (0, 0)),
            pl.BlockSpec((128, 8 * HID_C), lambda i: (0, 0)),
            pl.BlockSpec((8 * HID_C, 8 * OUT_C), lambda i: (0, 0)),
            pl.BlockSpec((128, 8 * OUT_C), lambda i: (0, 0)),
        ],
        out_specs=pl.BlockSpec((4, BR, 128), lambda i: (0, i, 0)),
        out_shape=jax.ShapeDtypeStruct((4, *LIN), jnp.float32),
    )(s1p, g1, dinv, b1g, bd1, bd2, rmat)

    s2 = _sc_prop64(src2d, dst2d, g2.reshape(4, NPAD, CW),
                    zeros_n).reshape(4, *LIN)

    b2lin = jnp.tile(b2.reshape(4, CW), (1, 8)).reshape(4, 1, 128)

    t4 = pl.pallas_call(
        _tc3_body,
        grid=(NPAD // BL1,),
        in_specs=[
            pl.BlockSpec((4, BR, 128), lambda i: (0, i, 0)),
            pl.BlockSpec((4, BR, 128), lambda i: (0, i, 0)),
            pl.BlockSpec((BR, 128), lambda i: (i, 0)),
            pl.BlockSpec((4, 1, 128), lambda i: (0, 0, 0)),
        ],
        out_specs=pl.BlockSpec((4, BR, 128), lambda i: (0, i, 0)),
        out_shape=jax.ShapeDtypeStruct((4, *LIN), jnp.float32),
    )(s2, g2, dinv, b2lin)

    out_t = t4.reshape(4, NPAD // 8, 8, CW).transpose(0, 3, 1, 2)
    out_t = out_t.reshape(4 * CW, NPAD)
    return out_t[:, :N_NODES].T
